# baseline (speedup 1.0000x reference)
import functools

import jax
import jax.numpy as jnp
from jax import lax
from jax.experimental import pallas as pl
from jax.experimental.pallas import tpu as pltpu
from jax.experimental.pallas import tpu_sc as plsc

C_LIGHT = 299792.458
CENTER_FREQ = 437100000.0

N = 1_000_000
N_PASSES = 100_000
NW = 32
CHUNK = 2048
NFULL = 15
PER_W = 31_264
TAIL = 544
VPC = CHUNK // 16
NIN = 5
NOUT = 2
TBL = N_PASSES + 1
DELTA_IDX = N_PASSES

_mesh = plsc.VectorSubcoreMesh(core_axis_name="c", subcore_axis_name="s")

_scratch = (
    [pltpu.VMEM_SHARED((TBL,), jnp.float32),
     pltpu.VMEM((TBL,), jnp.float32)]
    + [pltpu.VMEM((CHUNK,), jnp.int32) for _ in range(NIN)]
    + [pltpu.VMEM((CHUNK,), jnp.float32) for _ in range(NIN)]
    + [pltpu.VMEM((CHUNK,), jnp.float32) for _ in range(NOUT)]
    + [pltpu.SemaphoreType.DMA for _ in range(NIN + NOUT)]
)


@functools.partial(
    pl.kernel,
    out_type=jax.ShapeDtypeStruct((N,), jnp.float32),
    mesh=_mesh,
    compiler_params=pltpu.CompilerParams(needs_layout_passes=False),
    scratch_types=_scratch,
)
def _doppler_sc(rr_hbm, params_hbm, idx_hbm, out_hbm, table_sh, table_v, *bufs):
    idx_b = bufs[:NIN]
    rr_b = bufs[NIN:2 * NIN]
    out_b = bufs[2 * NIN:2 * NIN + NOUT]
    si_b = bufs[2 * NIN + NOUT:2 * NIN + NOUT + NIN]
    so_b = bufs[2 * NIN + NOUT + NIN:]

    wid = lax.axis_index("s") * 2 + lax.axis_index("c")
    base = jnp.minimum(wid * PER_W, N - PER_W)

    def start_in(g, size=CHUNK):
        b = g % NIN
        off = base + g * CHUNK
        pltpu.make_async_copy(
            idx_hbm.at[pl.ds(off, size)], idx_b[b].at[pl.ds(0, size)],
            si_b[b]).start()
        pltpu.make_async_copy(
            rr_hbm.at[pl.ds(off, size)], rr_b[b].at[pl.ds(0, size)],
            si_b[b]).start()

    def wait_in(g, size=CHUNK):
        b = g % NIN
        pltpu.make_async_copy(
            idx_hbm.at[pl.ds(base, size)], idx_b[b].at[pl.ds(0, size)],
            si_b[b]).wait()
        pltpu.make_async_copy(
            rr_hbm.at[pl.ds(base, size)], rr_b[b].at[pl.ds(0, size)],
            si_b[b]).wait()

    def start_out(g, size=CHUNK):
        b = g % NOUT
        off = base + g * CHUNK
        pltpu.make_async_copy(
            out_b[b].at[pl.ds(0, size)], out_hbm.at[pl.ds(off, size)],
            so_b[b]).start()

    def wait_out(g, size=CHUNK):
        b = g % NOUT
        pltpu.make_async_copy(
            out_b[b].at[pl.ds(0, size)], out_hbm.at[pl.ds(base, size)],
            so_b[b]).wait()

    for g in range(NIN):
        start_in(g)

    @pl.when(lax.axis_index("s") == 0)
    def _():
        pltpu.sync_copy(params_hbm, table_sh)

    plsc.subcore_barrier()
    pltpu.sync_copy(table_sh, table_v)

    didx = jnp.full((16,), DELTA_IDX, jnp.int32)
    delta = plsc.load_gather(table_v, [didx])
    scale = -(CENTER_FREQ + delta) / C_LIGHT

    def compute(g, nvec, unroll):
        ib, rb, ob = idx_b[g % NIN], rr_b[g % NIN], out_b[g % NOUT]

        def step(i):
            sl = pl.ds(pl.multiple_of(i * 16, 16), 16)
            bias = plsc.load_gather(table_v, [ib[sl]])
            ob[sl] = rb[sl] * scale + bias

        plsc.parallel_loop(0, nvec, 1, unroll=unroll)(step)

    for g in range(NFULL):
        wait_in(g)
        if g >= NOUT:
            wait_out(g)
        compute(g, VPC, 8)
        start_out(g)
        if g + NIN < NFULL:
            start_in(g + NIN)
        if g == NFULL - NIN:
            start_in(NFULL, TAIL)

    wait_in(NFULL, TAIL)
    wait_out(NFULL)
    compute(NFULL, TAIL // 16, 2)
    start_out(NFULL, TAIL)
    wait_out(NFULL - 1)
    wait_out(NFULL, TAIL)


def kernel(range_rate, sensor_params, contact_indices):
    idx32 = contact_indices.astype(jnp.int32)
    return _doppler_sc(range_rate, sensor_params, idx32)
